# baseline (speedup 1.0000x reference)
import jax
import jax.numpy as jnp
from jax import lax
from jax.experimental import pallas as pl
from jax.experimental.pallas import tpu as pltpu

_ROT0 = (13, 15, 26, 6)
_ROT1 = (17, 29, 16, 24)
_KS0 = 0
_KS1 = 42
_KS2 = _KS0 ^ _KS1 ^ 0x1BD11BDA

_BLOCK_S = 2048


def _rotl(x, d):
    return lax.shift_left(x, jnp.uint32(d)) | lax.shift_right_logical(
        x, jnp.uint32(32 - d)
    )


def _threefry2x32(x0, x1):
    def rounds(x0, x1, rots):
        for r in rots:
            x0 = x0 + x1
            x1 = _rotl(x1, r)
            x1 = x0 ^ x1
        return x0, x1

    x0 = x0 + jnp.uint32(_KS0)
    x1 = x1 + jnp.uint32(_KS1)
    x0, x1 = rounds(x0, x1, _ROT0)
    x0 = x0 + jnp.uint32(_KS1)
    x1 = x1 + jnp.uint32(_KS2 + 1)
    x0, x1 = rounds(x0, x1, _ROT1)
    x0 = x0 + jnp.uint32(_KS2)
    x1 = x1 + jnp.uint32(_KS0 + 2)
    x0, x1 = rounds(x0, x1, _ROT0)
    x0 = x0 + jnp.uint32(_KS0)
    x1 = x1 + jnp.uint32(_KS1 + 3)
    x0, x1 = rounds(x0, x1, _ROT1)
    x0 = x0 + jnp.uint32(_KS1)
    x1 = x1 + jnp.uint32(_KS2 + 4)
    x0, x1 = rounds(x0, x1, _ROT0)
    x0 = x0 + jnp.uint32(_KS2)
    x1 = x1 + jnp.uint32(_KS0 + 5)
    return x0, x1


def _gumbel_from_bits(bits):
    f = lax.bitcast_convert_type(
        lax.shift_right_logical(bits, jnp.uint32(9)) | jnp.uint32(0x3F800000),
        jnp.float32,
    ) - jnp.float32(1.0)
    u = jnp.maximum(f, jnp.float32(1e-20))
    return -jnp.log(-jnp.log(u))


def _make_body(block_s, vocab):
    def body(pred_ref, out_ref):
        i = pl.program_id(0)
        rows = lax.broadcasted_iota(jnp.int32, (block_s, vocab), 0)
        cols = lax.broadcasted_iota(jnp.int32, (block_s, vocab), 1)
        flat = ((i * block_s + rows) * vocab + cols).astype(jnp.uint32)
        b0, b1 = _threefry2x32(jnp.zeros_like(flat), flat)
        v = pred_ref[...] + _gumbel_from_bits(b0 ^ b1)
        res = jnp.argmax(v, axis=-1, keepdims=True).astype(jnp.int32)
        resb = jnp.broadcast_to(res, (block_s, 128))
        out_ref[...] = resb.T[0:1, :]

    return body


def kernel(pred):
    b, t, k = pred.shape
    n = b * t
    out = pl.pallas_call(
        _make_body(_BLOCK_S, k),
        grid=(n // _BLOCK_S,),
        in_specs=[pl.BlockSpec((_BLOCK_S, k), lambda i: (i, 0))],
        out_specs=pl.BlockSpec((1, _BLOCK_S), lambda i: (0, i)),
        out_shape=jax.ShapeDtypeStruct((1, n), jnp.int32),
        compiler_params=pltpu.CompilerParams(
            dimension_semantics=("parallel",)
        ),
    )(pred.reshape(n, k))
    return out.reshape(b, t)


# speedup vs baseline: 1.2413x; 1.2413x over previous
import jax
import jax.numpy as jnp
from jax import lax
from jax.experimental import pallas as pl
from jax.experimental.pallas import tpu as pltpu

_ROT0 = (13, 15, 26, 6)
_ROT1 = (17, 29, 16, 24)
_KS0 = 0
_KS1 = 42
_KS2 = _KS0 ^ _KS1 ^ 0x1BD11BDA

_BLOCK_S = 1024


def _rotl(x, d):
    return lax.shift_left(x, jnp.uint32(d)) | lax.shift_right_logical(
        x, jnp.uint32(32 - d)
    )


def _threefry2x32(x0, x1):
    def rounds(x0, x1, rots):
        for r in rots:
            x0 = x0 + x1
            x1 = _rotl(x1, r)
            x1 = x0 ^ x1
        return x0, x1

    x0 = x0 + jnp.uint32(_KS0)
    x1 = x1 + jnp.uint32(_KS1)
    x0, x1 = rounds(x0, x1, _ROT0)
    x0 = x0 + jnp.uint32(_KS1)
    x1 = x1 + jnp.uint32(_KS2 + 1)
    x0, x1 = rounds(x0, x1, _ROT1)
    x0 = x0 + jnp.uint32(_KS2)
    x1 = x1 + jnp.uint32(_KS0 + 2)
    x0, x1 = rounds(x0, x1, _ROT0)
    x0 = x0 + jnp.uint32(_KS0)
    x1 = x1 + jnp.uint32(_KS1 + 3)
    x0, x1 = rounds(x0, x1, _ROT1)
    x0 = x0 + jnp.uint32(_KS1)
    x1 = x1 + jnp.uint32(_KS2 + 4)
    x0, x1 = rounds(x0, x1, _ROT0)
    x0 = x0 + jnp.uint32(_KS2)
    x1 = x1 + jnp.uint32(_KS0 + 5)
    return x0, x1


def _gumbel_from_bits(bits):
    f = lax.bitcast_convert_type(
        lax.shift_right_logical(bits, jnp.uint32(9)) | jnp.uint32(0x3F800000),
        jnp.float32,
    ) - jnp.float32(1.0)
    u = jnp.maximum(f, jnp.float32(1e-20))
    return -jnp.log(-jnp.log(u))


def _make_body(block_s, vocab):
    def body(pred_ref, out_ref):
        i = pl.program_id(0)
        rows = lax.broadcasted_iota(jnp.int32, (block_s, vocab), 0)
        cols = lax.broadcasted_iota(jnp.int32, (block_s, vocab), 1)
        flat = ((i * block_s + rows) * vocab + cols).astype(jnp.uint32)
        b0, b1 = _threefry2x32(jnp.zeros_like(flat), flat)
        v = pred_ref[...] + _gumbel_from_bits(b0 ^ b1)
        res = jnp.argmax(v, axis=-1, keepdims=True).astype(jnp.int32)
        resb = jnp.broadcast_to(res, (block_s, 128))
        out_ref[...] = resb.T[0:1, :]

    return body


def kernel(pred):
    b, t, k = pred.shape
    n = b * t
    out = pl.pallas_call(
        _make_body(_BLOCK_S, k),
        grid=(n // _BLOCK_S,),
        in_specs=[pl.BlockSpec((_BLOCK_S, k), lambda i: (i, 0))],
        out_specs=pl.BlockSpec((1, _BLOCK_S), lambda i: (0, i)),
        out_shape=jax.ShapeDtypeStruct((1, n), jnp.int32),
        compiler_params=pltpu.CompilerParams(
            dimension_semantics=("parallel",)
        ),
    )(pred.reshape(n, k))
    return out.reshape(b, t)
